# Initial kernel scaffold; baseline (speedup 1.0000x reference)
import jax
import jax.numpy as jnp
from jax.experimental import pallas as pl


def kernel(adj_mat, degrees, maps, edge_index):
    raise NotImplementedError("write your pallas kernel here")


# speedup vs baseline: 30.7443x; 30.7443x over previous
import functools
import jax
import jax.numpy as jnp
from jax import lax
from jax.experimental import pallas as pl
from jax.experimental.pallas import tpu as pltpu
from jax.experimental.pallas import tpu_sc as plsc

NW = 32
CH = 64
_F32 = jnp.float32
_I32 = jnp.int32


def _iota():
    return lax.iota(_I32, 16)


def _splat_i(x):
    return jnp.full((16,), x, _I32)


def _rsqrt_newton(x):
    i = lax.bitcast_convert_type(x, _I32)
    i = jnp.int32(0x5F3759DF) - lax.shift_right_logical(i, 1)
    y = lax.bitcast_convert_type(i, _F32)
    for _ in range(4):
        y = y * (jnp.float32(1.5) - jnp.float32(0.5) * x * y * y)
    return y


def _vget(ref, idx):
    v = plsc.load_gather(ref, [_splat_i(idx)])
    return jnp.sum(jnp.where(_iota() == 0, v, 0))


def _load16(ref, base):
    return plsc.load_gather(ref, [_splat_i(base) + _iota()])


def _make_phase1(n_nodes, n_edges):
    E, N = n_edges, n_nodes
    EPW = E // NW
    GPW = 2 * E // NW
    PAD = CH * 16

    mesh = plsc.VectorSubcoreMesh(core_axis_name="c", subcore_axis_name="s")

    out_type = (
        jax.ShapeDtypeStruct((E * 16 + PAD,), _F32),
        jax.ShapeDtypeStruct((E * 16 + PAD,), _F32),
        jax.ShapeDtypeStruct((2 * E * 16 + PAD,), _F32),
        jax.ShapeDtypeStruct((6 * N,), _I32),
        jax.ShapeDtypeStruct((N,), _F32),
    )

    scratch = [
        pltpu.VMEM((N,), _F32),
        pltpu.VMEM((N,), _F32),
        pltpu.VMEM((EPW * 8,), _F32),
        pltpu.VMEM((EPW * 8,), _F32),
        pltpu.VMEM((EPW,), _I32),
        pltpu.VMEM((EPW,), _I32),
        pltpu.VMEM((EPW,), _F32),
        pltpu.VMEM((EPW * 16,), _F32),
        pltpu.VMEM((EPW * 16,), _F32),
        pltpu.VMEM((GPW * 8,), _F32),
        pltpu.VMEM((GPW * 16,), _F32),
        pltpu.VMEM((E + 16,), _I32),
        pltpu.VMEM((N + 32,), _I32),
        pltpu.VMEM((6 * N,), _I32),
    ]

    @functools.partial(pl.kernel, out_type=out_type, mesh=mesh,
                       scratch_types=scratch,
                       compiler_params=pltpu.CompilerParams(
                           needs_layout_passes=False))
    def phase1(maps_hbm, deg_hbm, r_hbm, c_hbm,
               b_out, bt_out, g_out, meta_out, dinv_out,
               degbuf, dinvbuf, fbuf, tbuf, rbuf, cbuf, sbuf,
               bloc, btloc, mbuf, gloc, rfull, endbuf, metabuf):
        wid = lax.axis_index("s") * 2 + lax.axis_index("c")
        iot = _iota()
        ia = lax.shift_right_logical(iot, 2)
        ib = lax.bitwise_and(iot, 3)

        pltpu.sync_copy(deg_hbm, degbuf)

        def dinv_body(k, _):
            x = _load16(degbuf, k * 16) * jnp.float32(2.0) + jnp.float32(1.0)
            plsc.store_scatter(dinvbuf, [_splat_i(k * 16) + iot],
                               _rsqrt_newton(x))
            return 0
        lax.fori_loop(0, N // 16, dinv_body, 0)

        e0 = wid * EPW
        pltpu.sync_copy(maps_hbm.at[pl.ds(e0 * 8, EPW * 8)], fbuf)
        pltpu.sync_copy(maps_hbm.at[pl.ds((E + e0) * 8, EPW * 8)], tbuf)
        pltpu.sync_copy(r_hbm.at[pl.ds(e0, EPW)], rbuf)
        pltpu.sync_copy(c_hbm.at[pl.ds(e0, EPW)], cbuf)

        def scale_body(k, _):
            rv = _load16(rbuf, k * 16)
            cv = _load16(cbuf, k * 16)
            dr = plsc.load_gather(dinvbuf, [rv])
            dc = plsc.load_gather(dinvbuf, [cv])
            plsc.store_scatter(sbuf, [_splat_i(k * 16) + iot], -(dr * dc))
            return 0
        lax.fori_loop(0, EPW // 16, scale_body, 0)

        def b_body(t, _):
            base = _splat_i(t * 8)
            fa0 = plsc.load_gather(fbuf, [base + ia])
            tb0 = plsc.load_gather(tbuf, [base + ib])
            fa1 = plsc.load_gather(fbuf, [base + 4 + ia])
            tb1 = plsc.load_gather(tbuf, [base + 4 + ib])
            s = plsc.load_gather(sbuf, [_splat_i(t)])
            blk = (fa0 * tb0 + fa1 * tb1) * s
            plsc.store_scatter(bloc, [_splat_i(t * 16) + iot], blk)
            plsc.store_scatter(btloc, [_splat_i(t * 16) + ib * 4 + ia], blk)
            return 0
        lax.fori_loop(0, EPW, b_body, 0)
        pltpu.sync_copy(bloc, b_out.at[pl.ds(e0 * 16, EPW * 16)])
        pltpu.sync_copy(btloc, bt_out.at[pl.ds(e0 * 16, EPW * 16)])

        g0 = wid * GPW
        pltpu.sync_copy(maps_hbm.at[pl.ds(g0 * 8, GPW * 8)], mbuf)

        def g_body(t, _):
            base = _splat_i(t * 8)
            ga0 = plsc.load_gather(mbuf, [base + ia])
            gb0 = plsc.load_gather(mbuf, [base + ib])
            ga1 = plsc.load_gather(mbuf, [base + 4 + ia])
            gb1 = plsc.load_gather(mbuf, [base + 4 + ib])
            plsc.store_scatter(gloc, [_splat_i(t * 16) + iot],
                               ga0 * gb0 + ga1 * gb1)
            return 0
        lax.fori_loop(0, GPW, g_body, 0)
        pltpu.sync_copy(gloc, g_out.at[pl.ds(g0 * 16, GPW * 16)])

        @pl.when(wid == 0)
        def _():
            pltpu.sync_copy(r_hbm, rfull.at[pl.ds(0, E)])
            plsc.store_scatter(rfull, [_splat_i(E) + iot], _splat_i(-1))

            def zend(k, _):
                plsc.store_scatter(endbuf, [_splat_i(k * 16) + iot],
                                   _splat_i(0))
                return 0
            lax.fori_loop(0, (N + 32) // 16, zend, 0)

            def bound(k, _):
                rv = _load16(rfull, k * 16)
                rn = _load16(rfull, k * 16 + 1)
                m = rv != rn
                plsc.store_scatter(endbuf, [rv + 1],
                                   _splat_i(k * 16 + 1) + iot, mask=m)
                return 0
            lax.fori_loop(0, E // 16, bound, 0)

            def fill(k, carry):
                v = _load16(endbuf, k * 16)
                cm = plsc.cummax(v)
                cm = jnp.maximum(cm, _splat_i(carry))
                plsc.store_scatter(endbuf, [_splat_i(k * 16) + iot], cm)
                return jnp.sum(jnp.where(iot == 15, cm, 0))
            lax.fori_loop(0, (N + 16) // 16, fill, 0)

            def meta_body(k, carry):
                cd, cl = carry
                degv = _load16(degbuf, k * 16).astype(_I32)
                usv = _load16(endbuf, k * 16)
                uev = _load16(endbuf, k * 16 + 1)
                udv = uev - usv
                ldv = degv - udv
                incd = plsc.cumsum(degv) + _splat_i(cd)
                incl = plsc.cumsum(ldv) + _splat_i(cl)
                off = _splat_i(k * 16) + iot
                plsc.store_scatter(metabuf, [off], incd - degv)
                plsc.store_scatter(metabuf, [_splat_i(N) + off], degv)
                plsc.store_scatter(metabuf, [_splat_i(2 * N) + off], usv)
                plsc.store_scatter(metabuf, [_splat_i(3 * N) + off], udv)
                plsc.store_scatter(metabuf, [_splat_i(4 * N) + off],
                                   incl - ldv)
                plsc.store_scatter(metabuf, [_splat_i(5 * N) + off], ldv)
                ncd = jnp.sum(jnp.where(iot == 15, incd, 0))
                ncl = jnp.sum(jnp.where(iot == 15, incl, 0))
                return (ncd, ncl)
            lax.fori_loop(0, N // 16, meta_body, (0, 0))
            pltpu.sync_copy(metabuf, meta_out)
            pltpu.sync_copy(dinvbuf, dinv_out)

    return phase1


def _make_phase2(n_nodes, n_edges):
    E, N = n_edges, n_nodes
    RPW = N // NW

    mesh = plsc.VectorSubcoreMesh(core_axis_name="c", subcore_axis_name="s")
    out_type = jax.ShapeDtypeStruct((4 * N, 4 * N), _F32)

    scratch = [
        pltpu.VMEM((6 * N,), _I32),
        pltpu.VMEM((N,), _F32),
        pltpu.VMEM((N,), _F32),
        pltpu.VMEM((CH * 16,), _F32),
        pltpu.VMEM((CH * 16,), _F32),
        pltpu.VMEM((CH * 16,), _F32),
        pltpu.VMEM((N + 16,), _I32),
        pltpu.VMEM((4, 4 * N), _F32),
    ]

    @functools.partial(pl.kernel, out_type=out_type, mesh=mesh,
                       scratch_types=scratch,
                       compiler_params=pltpu.CompilerParams(
                           needs_layout_passes=False))
    def phase2(adj_hbm, b_hbm, bt_hbm, g_hbm, meta_hbm, dinv_hbm, out_hbm,
               metabuf, dinvbuf, adjrow, gch, bch, btch, collist, strip):
        wid = lax.axis_index("s") * 2 + lax.axis_index("c")
        iot = _iota()
        ia = lax.shift_right_logical(iot, 2)
        ib = lax.bitwise_and(iot, 3)
        zf = jnp.zeros((16,), _F32)

        pltpu.sync_copy(meta_hbm, metabuf)
        pltpu.sync_copy(dinv_hbm, dinvbuf)

        def z_body(k, _):
            plsc.store_scatter(
                strip,
                [_splat_i(lax.shift_right_logical(k, 8)),
                 _splat_i(lax.bitwise_and(k, 255) * 16) + iot],
                zf)
            return 0
        lax.fori_loop(0, N, z_body, 0)

        def row_body(t, carry):
            cnt_prev, prev_i = carry
            i = wid * RPW + t

            @pl.when(prev_i >= 0)
            def _():
                def undo(u, _):
                    colspl = plsc.load_gather(collist, [_splat_i(u)])
                    plsc.store_scatter(strip, [ia, colspl * 4 + ib], zf)
                    return 0
                lax.fori_loop(0, cnt_prev, undo, 0)
                plsc.store_scatter(strip, [ia, _splat_i(prev_i * 4) + ib], zf)

            pltpu.sync_copy(adj_hbm.at[pl.ds(i * N, N)], adjrow)

            def scan(k, cnt):
                av = _load16(adjrow, k * 16)
                m = av != jnp.float32(0.0)
                mi = m.astype(_I32)
                cs = plsc.cumsum(mi)
                pos = _splat_i(cnt) + cs - 1
                plsc.store_scatter(collist, [pos],
                                   _splat_i(k * 16) + iot, mask=m)
                return cnt + jnp.sum(mi)
            lax.fori_loop(0, N // 16, scan, 0)

            rs = _vget(metabuf, i)
            dg = _vget(metabuf, N + i)
            us_ = _vget(metabuf, 2 * N + i)
            ud = _vget(metabuf, 3 * N + i)
            ls_ = _vget(metabuf, 4 * N + i)
            ld = _vget(metabuf, 5 * N + i)

            dspl = plsc.load_gather(dinvbuf, [_splat_i(i)])

            def diag_chunk(j, acc):
                pltpu.sync_copy(g_hbm.at[pl.ds((rs + j * CH) * 16, CH * 16)],
                                gch)
                lim = jnp.minimum(CH, dg - j * CH)

                def dsum(tt, a):
                    return a + _load16(gch, tt * 16)
                return lax.fori_loop(0, lim, dsum, acc)
            acc = lax.fori_loop(0, (dg + CH - 1) // CH, diag_chunk, zf)
            plsc.store_scatter(strip, [ia, _splat_i(i * 4) + ib],
                               acc * dspl * dspl)

            def low_chunk(j, _):
                pltpu.sync_copy(
                    bt_hbm.at[pl.ds((ls_ + j * CH) * 16, CH * 16)], btch)
                lim = jnp.minimum(CH, ld - j * CH)

                def place(tt, __):
                    colspl = plsc.load_gather(collist,
                                              [_splat_i(j * CH + tt)])
                    blk = _load16(btch, tt * 16)
                    plsc.store_scatter(strip, [ia, colspl * 4 + ib], blk)
                    return 0
                lax.fori_loop(0, lim, place, 0)
                return 0
            lax.fori_loop(0, (ld + CH - 1) // CH, low_chunk, 0)

            def up_chunk(j, _):
                pltpu.sync_copy(
                    b_hbm.at[pl.ds((us_ + j * CH) * 16, CH * 16)], bch)
                lim = jnp.minimum(CH, ud - j * CH)

                def place(tt, __):
                    colspl = plsc.load_gather(collist,
                                              [_splat_i(ld + j * CH + tt)])
                    blk = _load16(bch, tt * 16)
                    plsc.store_scatter(strip, [ia, colspl * 4 + ib], blk)
                    return 0
                lax.fori_loop(0, lim, place, 0)
                return 0
            lax.fori_loop(0, (ud + CH - 1) // CH, up_chunk, 0)

            pltpu.sync_copy(strip, out_hbm.at[pl.ds(i * 4, 4)])
            return (dg, i)

        lax.fori_loop(0, RPW, row_body, (0, -1))

    return phase2


def kernel(adj_mat, degrees, maps, edge_index):
    N = adj_mat.shape[0]
    E = maps.shape[0] // 2
    assert N % NW == 0 and E % (16 * NW) == 0 and maps.shape[1:] == (2, 4)

    maps_flat = maps.reshape(-1).astype(_F32)
    adj_flat = adj_mat.reshape(-1).astype(_F32)
    r_e = edge_index[0, :E].astype(_I32)
    c_e = edge_index[1, :E].astype(_I32)
    deg = degrees.astype(_F32)

    b, bt, g, meta, dinv = _make_phase1(N, E)(maps_flat, deg, r_e, c_e)
    return _make_phase2(N, E)(adj_flat, b, bt, g, meta, dinv)
